# baseline (speedup 1.0000x reference)
import functools

import jax
import jax.numpy as jnp
from jax import lax
from jax.experimental import pallas as pl
from jax.experimental.pallas import tpu as pltpu
from jax.experimental.pallas import tpu_sc as plsc

EPS = 1e-7
D = 128
F = 384


def _col0_mask(c):
    return lax.broadcasted_iota(jnp.int32, (1, c), 1) == 0


def _zero_col0(a):
    return jnp.where(_col0_mask(a.shape[-1]), 0.0, a)


def _mdot(a, b):
    return jnp.sum(a * b, axis=-1, keepdims=True) - 2.0 * a[:, 0:1] * b[:, 0:1]


def _acosh(z):
    z = jnp.maximum(z, 1.0 + EPS)
    return jnp.log(z + jnp.sqrt((z - 1.0) * (z + 1.0)))


def _spn2(a):
    s = jnp.sum(a * a, axis=-1, keepdims=True) - a[:, 0:1] * a[:, 0:1]
    return jnp.maximum(s, 0.0)


def _logmap0(xp):
    d = _acosh(xp[:, 0:1])
    spn = jnp.sqrt(_spn2(xp) + 1e-15)
    return _zero_col0(xp * (d / spn))


def _expmap0(u):
    nrm = jnp.sqrt(_spn2(u) + 1e-15)
    e = jnp.exp(nrm)
    ei = 1.0 / e
    c = 0.5 * (e + ei)
    s = 0.5 * (e - ei) / nrm
    return jnp.where(_col0_mask(u.shape[-1]), c, u * s)


def _transp0(xp, u):
    f = _mdot(xp, u) / (1.0 + xp[:, 0:1])
    return u + f * xp + jnp.where(_col0_mask(xp.shape[-1]), f, 0.0)


def _expmap(xp, u):
    un = jnp.sqrt(jnp.maximum(_mdot(u, u), 1e-8))
    e = jnp.exp(un)
    ei = 1.0 / e
    return 0.5 * (e + ei) * xp + (0.5 * (e - ei) / un) * u


def _silu(z):
    return z / (1.0 + jnp.exp(-z))


def _sigmoid(z):
    return 1.0 / (1.0 + jnp.exp(-z))


def _node_pre_body(xf_ref, wlt_ref, bias_ref, t_ref, xtan_ref):
    xf = xf_ref[...]
    h = jnp.concatenate(
        [_logmap0(xf[:, :D]), _logmap0(xf[:, D:2 * D]), xf[:, 2 * D:]], axis=-1)
    h = jnp.dot(h, wlt_ref[...], preferred_element_type=jnp.float32)
    h = _zero_col0(h)
    x0p = _expmap0(h[:, :D])
    x1p = _expmap0(h[:, D:2 * D])
    x2 = h[:, 2 * D:]
    bias = _zero_col0(bias_ref[...])
    x0p = _expmap(x0p, _transp0(x0p, bias[:, :D]))
    x1p = _expmap(x1p, _transp0(x1p, bias[:, D:2 * D]))
    x2 = x2 + bias[:, 2 * D:]
    t_ref[...] = jnp.concatenate([x0p, x1p, x2], axis=-1)
    xtan_ref[...] = jnp.concatenate(
        [_logmap0(x0p), _logmap0(x1p), x2], axis=-1)


def _logmap0_h(xp):
    t = xp[:, 0:1]
    d = _acosh(t)
    spn = jnp.sqrt(jnp.maximum(t * t - 1.0, 0.0) + 1e-15)
    return xp * (d / spn)


def _edge_body(r_ref, c_ref, eattr_ref, emask_ref, ob_ref, wa1_ref, ba1_ref,
               wa2_ref, ba2_ref, we1_ref, be1_ref, we2_ref, be2_ref, wmsg_ref,
               ea_ref):
    rb = r_ref[...]
    cb = c_ref[...]
    x0r, x1r, x2r = rb[:, :D], rb[:, D:2 * D], rb[:, 2 * D:]
    x0c, x1c, x2c = cb[:, :D], cb[:, D:2 * D], cb[:, 2 * D:]
    x0rz = _zero_col0(x0r)
    x1rz = _zero_col0(x1r)
    x0cz = _zero_col0(x0c)
    x1cz = _zero_col0(x1c)
    prods = jnp.concatenate(
        [x0rz * x0c, x1rz * x1c, x0rz * x0rz, x1rz * x1rz, x0cz * x0cz,
         x1cz * x1cz], axis=-1)
    sums = jnp.dot(prods, ob_ref[...], preferred_element_type=jnp.float32,
                   precision=lax.Precision.HIGHEST)
    md0 = sums[:, 0:1] - x0r[:, 0:1] * x0c[:, 0:1]
    md1 = sums[:, 1:2] - x1r[:, 0:1] * x1c[:, 0:1]
    geo0 = _acosh(-md0)
    geo1 = _acosh(-md1)
    em = emask_ref[...]
    ea = jnp.concatenate([eattr_ref[...], geo0, geo1], axis=-1)
    distm = ea * em

    def lm(xp, s):
        d = _acosh(xp[:, 0:1])
        return xp * (d / jnp.sqrt(s + 1e-15))

    xtr = jnp.concatenate(
        [lm(x0r, sums[:, 2:3]), lm(x1r, sums[:, 3:4]), x2r], axis=-1)
    xtc = jnp.concatenate(
        [lm(x0c, sums[:, 4:5]), lm(x1c, sums[:, 5:6]), x2c], axis=-1)
    att_in = jnp.concatenate([xtr, xtc, distm], axis=-1)
    a1 = _silu(jnp.dot(att_in, wa1_ref[...],
                       preferred_element_type=jnp.float32) + ba1_ref[...])
    att = _sigmoid(jnp.dot(a1, wa2_ref[...],
                           preferred_element_type=jnp.float32)
                   + ba2_ref[...]) * em

    u0 = x0c + md0 * x0r
    u1 = x1c + md1 * x1r
    u0z = _zero_col0(u0)
    u1z = _zero_col0(u1)
    uprods = jnp.concatenate([u0z * u0z, u1z * u1z], axis=-1)
    usums = jnp.dot(uprods, ob_ref[:2 * D, :2],
                    preferred_element_type=jnp.float32,
                    precision=lax.Precision.HIGHEST)
    uu0 = usums[:, 0:1] - u0[:, 0:1] * u0[:, 0:1]
    uu1 = usums[:, 1:2] - u1[:, 0:1] * u1[:, 0:1]

    def mcalc(xr, u, uu, geo):
        un = jnp.sqrt(jnp.maximum(uu, 1e-8))
        mu = (geo / un) * u
        cc = -mu[:, 0:1] / (1.0 + xr[:, 0:1])
        return mu + cc * xr + jnp.where(_col0_mask(D), cc, 0.0)

    m0 = mcalc(x0r, u0, uu0, geo0)
    m1 = mcalc(x1r, u1, uu1, geo1)
    m2 = x2c - x2r
    mi = jnp.concatenate([m0, m1, m2, ea], axis=-1)
    mm = _silu(jnp.dot(mi, we1_ref[...],
                       preferred_element_type=jnp.float32) + be1_ref[...])
    mm = jnp.dot(mm, we2_ref[...],
                 preferred_element_type=jnp.float32) + be2_ref[...]
    wmsg_ref[...] = jnp.concatenate(
        [mm[:, :D] * att[:, 0:1], mm[:, D:2 * D] * att[:, 1:2],
         mm[:, 2 * D:] * att[:, 2:3]], axis=-1)
    ea_ref[...] = ea


def _node_post_body(t_ref, xtan_ref, agga_ref, aggb_ref, wn1_ref, bn1_ref,
                    wn2_ref, bn2_ref, lng_ref, lnb_ref, out_ref):
    tb = t_ref[...]
    x0p, x1p, x2 = tb[:, :D], tb[:, D:2 * D], tb[:, 2 * D:]
    ab = aggb_ref[...]
    h = jnp.concatenate(
        [xtan_ref[...], agga_ref[...], ab[:, :D] + ab[:, D:]], axis=-1)
    h = _silu(jnp.dot(h, wn1_ref[...],
                      preferred_element_type=jnp.float32) + bn1_ref[...])
    h = jnp.dot(h, wn2_ref[...],
                preferred_element_type=jnp.float32) + bn2_ref[...]
    h = _zero_col0(h)
    x0n = _expmap(x0p, _transp0(x0p, h[:, :D]))
    x1n = _expmap(x1p, _transp0(x1p, h[:, D:2 * D]))
    x2n = x2 + h[:, 2 * D:]
    xc = jnp.concatenate([_logmap0(x0n), _logmap0(x1n), x2n], axis=-1)
    t0 = xc[:, 0:1]
    mu = (jnp.sum(xc, axis=-1, keepdims=True) - t0) / (F - 1.0)
    dd = xc - mu
    d0 = t0 - mu
    var = (jnp.sum(dd * dd, axis=-1, keepdims=True) - d0 * d0) / (F - 1.0)
    tail = dd * lax.rsqrt(var + 1e-5) * lng_ref[...] + lnb_ref[...]
    xc = jnp.where(_col0_mask(F), xc, tail)
    xc = _zero_col0(_silu(xc))
    out_ref[...] = jnp.concatenate(
        [_expmap0(xc[:, :D]), _expmap0(xc[:, D:2 * D]), xc[:, 2 * D:]],
        axis=-1)


def _sc_gather(table, row, col):
    e = row.shape[0]
    fw = table.shape[1]
    nw = 32
    per = e // nw
    ch = 128
    nfull = per // ch
    rem = per - nfull * ch
    mesh = plsc.VectorSubcoreMesh(core_axis_name="c", subcore_axis_name="s")

    @functools.partial(
        pl.kernel, mesh=mesh,
        out_type=(jax.ShapeDtypeStruct((e, fw), jnp.float32),
                  jax.ShapeDtypeStruct((e, fw), jnp.float32)),
        scratch_types=[
            pltpu.VMEM((ch,), jnp.int32),
            pltpu.VMEM((ch, fw), jnp.float32),
            pltpu.VMEM((16,), jnp.int32),
            pltpu.VMEM((16, fw), jnp.float32),
            pltpu.SemaphoreType.DMA,
        ],
    )
    def gk(tab_h, row_h, col_h, r_h, c_h, idx_v, buf_v, idxs_v, bufs_v, sem):
        wid = lax.axis_index("s") * 2 + lax.axis_index("c")
        base = wid * per

        def do(idx_h, out_h):
            def body(j, carry):
                off = base + j * ch
                pltpu.sync_copy(idx_h.at[pl.ds(off, ch)], idx_v)
                pltpu.async_copy(tab_h.at[idx_v], buf_v, sem).wait()
                pltpu.sync_copy(buf_v, out_h.at[pl.ds(off, ch)])
                return carry
            lax.fori_loop(0, nfull, body, 0)
            if rem:
                off = base + per - 16
                pltpu.sync_copy(idx_h.at[pl.ds(off, 16)], idxs_v)
                pltpu.async_copy(tab_h.at[idxs_v], bufs_v, sem).wait()
                pltpu.sync_copy(bufs_v, out_h.at[pl.ds(off, 16)])

        do(row_h, r_h)
        do(col_h, c_h)

    return gk(table, row, col)


def _scatter_chunks(row_h, w_h, acc_s, idx_v, buf_v, idxt_v, buft_v,
                    base, ch, nfull, tail_cond, coloff):
    def sc(j, carry):
        off = base + j * ch
        pltpu.sync_copy(row_h.at[pl.ds(off, ch)], idx_v)
        pltpu.sync_copy(w_h.at[pl.ds(off, ch), pl.ds(coloff, D)], buf_v)
        pltpu.sync_copy(buf_v, acc_s.at[idx_v], add=True)
        return carry
    lax.fori_loop(0, nfull, sc, 0)

    @pl.when(tail_cond)
    def _():
        off = base + nfull * ch
        pltpu.sync_copy(row_h.at[pl.ds(off, 16)], idxt_v)
        pltpu.sync_copy(w_h.at[pl.ds(off, 16), pl.ds(coloff, D)], buft_v)
        pltpu.sync_copy(buft_v, acc_s.at[idxt_v], add=True)


def _zero_acc(z_h, blk_v, acc_s, s, zr, nzb, nzi):
    pltpu.sync_copy(z_h, blk_v)

    def zb(j, carry):
        b = j * 16 + s

        @pl.when(b < nzb)
        def _():
            pltpu.sync_copy(blk_v, acc_s.at[pl.ds(b * zr, zr)])
        return carry
    lax.fori_loop(0, nzi, zb, 0)


def _writeout(acc_s, out_h, blk_v, s, zr, nzb, nzi, coloff):
    def wb(j, carry):
        b = j * 16 + s

        @pl.when(b < nzb)
        def _():
            pltpu.sync_copy(acc_s.at[pl.ds(b * zr, zr)], blk_v)
            pltpu.sync_copy(
                blk_v, out_h.at[pl.ds(b * zr, zr), pl.ds(coloff, D)])
        return carry
    lax.fori_loop(0, nzi, wb, 0)


def _sc_scatter(wmsg, row, nnodes):
    e, fw = wmsg.shape
    ch = 128
    zr = 8
    nzb = nnodes // zr
    nzi = (nzb + 15) // 16
    zeros = jnp.zeros((zr, D), jnp.float32)
    mesh = plsc.VectorSubcoreMesh(core_axis_name="c", subcore_axis_name="s")
    scratch = [
        pltpu.VMEM((ch,), jnp.int32),
        pltpu.VMEM((ch, D), jnp.float32),
        pltpu.VMEM((16,), jnp.int32),
        pltpu.VMEM((16, D), jnp.float32),
        pltpu.VMEM((zr, D), jnp.float32),
        pltpu.VMEM_SHARED((nnodes, D), jnp.float32),
    ]

    pert1 = e // 16
    nf1 = pert1 // ch

    @functools.partial(
        pl.kernel, mesh=mesh,
        out_type=jax.ShapeDtypeStruct((nnodes, 2 * D), jnp.float32),
        scratch_types=scratch,
    )
    def sk1(w_h, row_h, z_h, agg_h, idx_v, buf_v, idxt_v, buft_v, blk_v,
            acc_s):
        c = lax.axis_index("c")
        s = lax.axis_index("s")
        _zero_acc(z_h, blk_v, acc_s, s, zr, nzb, nzi)
        plsc.subcore_barrier()
        _scatter_chunks(row_h, w_h, acc_s, idx_v, buf_v, idxt_v, buft_v,
                        s * pert1, ch, nf1, s >= 0, c * D)
        plsc.subcore_barrier()
        _writeout(acc_s, agg_h, blk_v, s, zr, nzb, nzi, c * D)

    nf2 = (e // 32) // ch

    @functools.partial(
        pl.kernel, mesh=mesh,
        out_type=jax.ShapeDtypeStruct((nnodes, 2 * D), jnp.float32),
        scratch_types=scratch,
    )
    def sk2(w_h, row_h, z_h, agg_h, idx_v, buf_v, idxt_v, buft_v, blk_v,
            acc_s):
        c = lax.axis_index("c")
        s = lax.axis_index("s")
        wid = c * 16 + s
        base = wid * (nf2 * ch) + jnp.minimum(wid, 16) * 16
        _zero_acc(z_h, blk_v, acc_s, s, zr, nzb, nzi)
        plsc.subcore_barrier()
        _scatter_chunks(row_h, w_h, acc_s, idx_v, buf_v, idxt_v, buft_v,
                        base, ch, nf2, wid < 16, 2 * D)
        plsc.subcore_barrier()
        _writeout(acc_s, agg_h, blk_v, s, zr, nzb, nzi, c * D)

    agg_a = sk1(wmsg, row, zeros)
    agg_b = sk2(wmsg, row, zeros)
    return agg_a, agg_b


def kernel(x, edge_attr, edges, node_mask, edge_mask, W_lin, bias, W_e1, b_e1,
           W_e2, b_e2, W_n1, b_n1, W_n2, b_n2, W_a1, b_a1, W_a2, b_a2,
           ln_g, ln_b):
    n = x.shape[0]
    e = edge_attr.shape[0]
    xf = x.reshape(n, F)
    bn = 1000

    t_tab, xtan = pl.pallas_call(
        _node_pre_body,
        grid=(n // bn,),
        in_specs=[
            pl.BlockSpec((bn, F), lambda i: (i, 0)),
            pl.BlockSpec((F, F), lambda i: (0, 0)),
            pl.BlockSpec((1, F), lambda i: (0, 0)),
        ],
        out_specs=[pl.BlockSpec((bn, F), lambda i: (i, 0))] * 2,
        out_shape=[jax.ShapeDtypeStruct((n, F), jnp.float32)] * 2,
    )(xf, W_lin.T, bias)

    row = edges[0]
    col = edges[1]
    r_tab, c_tab = _sc_gather(t_tab, row, col)

    ob = (jnp.arange(6 * D)[:, None] // D
          == jnp.arange(6)[None, :]).astype(jnp.float32)
    wa1t = W_a1.T
    wa1t = wa1t.at[jnp.array([0, D, F, F + D])].set(0.0)

    be = 640
    wmsg, ea = pl.pallas_call(
        _edge_body,
        grid=(e // be,),
        in_specs=[
            pl.BlockSpec((be, F), lambda i: (i, 0)),
            pl.BlockSpec((be, F), lambda i: (i, 0)),
            pl.BlockSpec((be, 2), lambda i: (i, 0)),
            pl.BlockSpec((be, 1), lambda i: (i, 0)),
            pl.BlockSpec((6 * D, 6), lambda i: (0, 0)),
            pl.BlockSpec((2 * F + 4, F), lambda i: (0, 0)),
            pl.BlockSpec((1, F), lambda i: (0, 0)),
            pl.BlockSpec((F, 3), lambda i: (0, 0)),
            pl.BlockSpec((1, 3), lambda i: (0, 0)),
            pl.BlockSpec((F + 4, F), lambda i: (0, 0)),
            pl.BlockSpec((1, F), lambda i: (0, 0)),
            pl.BlockSpec((F, F), lambda i: (0, 0)),
            pl.BlockSpec((1, F), lambda i: (0, 0)),
        ],
        out_specs=[
            pl.BlockSpec((be, F), lambda i: (i, 0)),
            pl.BlockSpec((be, 4), lambda i: (i, 0)),
        ],
        out_shape=[
            jax.ShapeDtypeStruct((e, F), jnp.float32),
            jax.ShapeDtypeStruct((e, 4), jnp.float32),
        ],
    )(r_tab, c_tab, edge_attr, edge_mask, ob, wa1t, b_a1.reshape(1, F),
      W_a2.T, b_a2.reshape(1, 3), W_e1.T, b_e1.reshape(1, F), W_e2.T,
      b_e2.reshape(1, F))

    agg_a, agg_b = _sc_scatter(wmsg, row, n)

    lng = jnp.concatenate([jnp.zeros((1,), jnp.float32), ln_g]).reshape(1, F)
    lnb = jnp.concatenate([jnp.zeros((1,), jnp.float32), ln_b]).reshape(1, F)
    out = pl.pallas_call(
        _node_post_body,
        grid=(n // bn,),
        in_specs=[
            pl.BlockSpec((bn, F), lambda i: (i, 0)),
            pl.BlockSpec((bn, F), lambda i: (i, 0)),
            pl.BlockSpec((bn, 2 * D), lambda i: (i, 0)),
            pl.BlockSpec((bn, 2 * D), lambda i: (i, 0)),
            pl.BlockSpec((2 * F, F), lambda i: (0, 0)),
            pl.BlockSpec((1, F), lambda i: (0, 0)),
            pl.BlockSpec((F, F), lambda i: (0, 0)),
            pl.BlockSpec((1, F), lambda i: (0, 0)),
            pl.BlockSpec((1, F), lambda i: (0, 0)),
            pl.BlockSpec((1, F), lambda i: (0, 0)),
        ],
        out_specs=[pl.BlockSpec((bn, F), lambda i: (i, 0))],
        out_shape=[jax.ShapeDtypeStruct((n, F), jnp.float32)],
    )(t_tab, xtan, agg_a, agg_b, W_n1.T, b_n1.reshape(1, F), W_n2.T,
      b_n2.reshape(1, F), lng, lnb)[0]

    return out.reshape(n, 3, D), ea, edges, node_mask, edge_mask


# speedup vs baseline: 1.5301x; 1.5301x over previous
import functools

import jax
import jax.numpy as jnp
from jax import lax
from jax.experimental import pallas as pl
from jax.experimental.pallas import tpu as pltpu
from jax.experimental.pallas import tpu_sc as plsc

EPS = 1e-7
D = 128
F = 384


def _col0_mask(c):
    return lax.broadcasted_iota(jnp.int32, (1, c), 1) == 0


def _zero_col0(a):
    return jnp.where(_col0_mask(a.shape[-1]), 0.0, a)


def _mdot(a, b):
    return jnp.sum(a * b, axis=-1, keepdims=True) - 2.0 * a[:, 0:1] * b[:, 0:1]


def _acosh(z):
    z = jnp.maximum(z, 1.0 + EPS)
    return jnp.log(z + jnp.sqrt((z - 1.0) * (z + 1.0)))


def _spn2(a):
    s = jnp.sum(a * a, axis=-1, keepdims=True) - a[:, 0:1] * a[:, 0:1]
    return jnp.maximum(s, 0.0)


def _logmap0(xp):
    d = _acosh(xp[:, 0:1])
    spn = jnp.sqrt(_spn2(xp) + 1e-15)
    return _zero_col0(xp * (d / spn))


def _expmap0(u):
    nrm = jnp.sqrt(_spn2(u) + 1e-15)
    e = jnp.exp(nrm)
    ei = 1.0 / e
    c = 0.5 * (e + ei)
    s = 0.5 * (e - ei) / nrm
    return jnp.where(_col0_mask(u.shape[-1]), c, u * s)


def _transp0(xp, u):
    f = _mdot(xp, u) / (1.0 + xp[:, 0:1])
    return u + f * xp + jnp.where(_col0_mask(xp.shape[-1]), f, 0.0)


def _expmap(xp, u):
    un = jnp.sqrt(jnp.maximum(_mdot(u, u), 1e-8))
    e = jnp.exp(un)
    ei = 1.0 / e
    return 0.5 * (e + ei) * xp + (0.5 * (e - ei) / un) * u


def _silu(z):
    return z / (1.0 + jnp.exp(-z))


def _sigmoid(z):
    return 1.0 / (1.0 + jnp.exp(-z))


def _node_pre_body(xf_ref, wlt_ref, bias_ref, t_ref, xtan_ref):
    xf = xf_ref[...]
    h = jnp.concatenate(
        [_logmap0(xf[:, :D]), _logmap0(xf[:, D:2 * D]), xf[:, 2 * D:]], axis=-1)
    h = jnp.dot(h, wlt_ref[...], preferred_element_type=jnp.float32)
    h = _zero_col0(h)
    x0p = _expmap0(h[:, :D])
    x1p = _expmap0(h[:, D:2 * D])
    x2 = h[:, 2 * D:]
    bias = _zero_col0(bias_ref[...])
    x0p = _expmap(x0p, _transp0(x0p, bias[:, :D]))
    x1p = _expmap(x1p, _transp0(x1p, bias[:, D:2 * D]))
    x2 = x2 + bias[:, 2 * D:]
    t_ref[...] = jnp.concatenate([x0p, x1p, x2], axis=-1)
    xtan_ref[...] = jnp.concatenate(
        [_logmap0(x0p), _logmap0(x1p), x2], axis=-1)


def _logmap0_h(xp):
    t = xp[:, 0:1]
    d = _acosh(t)
    spn = jnp.sqrt(jnp.maximum(t * t - 1.0, 0.0) + 1e-15)
    return xp * (d / spn)


def _edge_body(r_ref, c_ref, eattr_ref, emask_ref, wa1_ref, ba1_ref,
               wa2_ref, ba2_ref, we1_ref, be1_ref, we2_ref, be2_ref, wmsg_ref,
               ea_ref):
    rb = r_ref[...]
    cb = c_ref[...]
    x0r, x1r, x2r = rb[:, :D], rb[:, D:2 * D], rb[:, 2 * D:]
    x0c, x1c, x2c = cb[:, :D], cb[:, D:2 * D], cb[:, 2 * D:]
    md0 = _mdot(x0r, x0c)
    md1 = _mdot(x1r, x1c)
    geo0 = _acosh(-md0)
    geo1 = _acosh(-md1)
    em = emask_ref[...]
    ea = jnp.concatenate([eattr_ref[...], geo0, geo1], axis=-1)
    distm = ea * em
    xtr = jnp.concatenate([_logmap0_h(x0r), _logmap0_h(x1r), x2r], axis=-1)
    xtc = jnp.concatenate([_logmap0_h(x0c), _logmap0_h(x1c), x2c], axis=-1)
    att_in = jnp.concatenate([xtr, xtc, distm], axis=-1)
    a1 = _silu(jnp.dot(att_in, wa1_ref[...],
                       preferred_element_type=jnp.float32) + ba1_ref[...])
    att = _sigmoid(jnp.dot(a1, wa2_ref[...],
                           preferred_element_type=jnp.float32)
                   + ba2_ref[...]) * em

    u0 = x0c + md0 * x0r
    u1 = x1c + md1 * x1r
    uu0 = md0 * md0 - 1.0
    uu1 = md1 * md1 - 1.0

    def mcalc(xr, u, uu, geo):
        un = jnp.sqrt(jnp.maximum(uu, 1e-8))
        mu = (geo / un) * u
        cc = -mu[:, 0:1] / (1.0 + xr[:, 0:1])
        return mu + cc * xr + jnp.where(_col0_mask(D), cc, 0.0)

    m0 = mcalc(x0r, u0, uu0, geo0)
    m1 = mcalc(x1r, u1, uu1, geo1)
    m2 = x2c - x2r
    mi = jnp.concatenate([m0, m1, m2, ea], axis=-1)
    mm = _silu(jnp.dot(mi, we1_ref[...],
                       preferred_element_type=jnp.float32) + be1_ref[...])
    mm = jnp.dot(mm, we2_ref[...],
                 preferred_element_type=jnp.float32) + be2_ref[...]
    wmsg_ref[...] = jnp.concatenate(
        [mm[:, :D] * att[:, 0:1], mm[:, D:2 * D] * att[:, 1:2],
         mm[:, 2 * D:] * att[:, 2:3]], axis=-1)
    ea_ref[...] = ea


def _node_post_body(t_ref, xtan_ref, agga_ref, aggb_ref, wn1_ref, bn1_ref,
                    wn2_ref, bn2_ref, lng_ref, lnb_ref, out_ref):
    tb = t_ref[...]
    x0p, x1p, x2 = tb[:, :D], tb[:, D:2 * D], tb[:, 2 * D:]
    ab = aggb_ref[...]
    h = jnp.concatenate(
        [xtan_ref[...], agga_ref[...], ab[:, :D] + ab[:, D:]], axis=-1)
    h = _silu(jnp.dot(h, wn1_ref[...],
                      preferred_element_type=jnp.float32) + bn1_ref[...])
    h = jnp.dot(h, wn2_ref[...],
                preferred_element_type=jnp.float32) + bn2_ref[...]
    h = _zero_col0(h)
    x0n = _expmap(x0p, _transp0(x0p, h[:, :D]))
    x1n = _expmap(x1p, _transp0(x1p, h[:, D:2 * D]))
    x2n = x2 + h[:, 2 * D:]
    xc = jnp.concatenate([_logmap0(x0n), _logmap0(x1n), x2n], axis=-1)
    t0 = xc[:, 0:1]
    mu = (jnp.sum(xc, axis=-1, keepdims=True) - t0) / (F - 1.0)
    dd = xc - mu
    d0 = t0 - mu
    var = (jnp.sum(dd * dd, axis=-1, keepdims=True) - d0 * d0) / (F - 1.0)
    tail = dd * lax.rsqrt(var + 1e-5) * lng_ref[...] + lnb_ref[...]
    xc = jnp.where(_col0_mask(F), xc, tail)
    xc = _zero_col0(_silu(xc))
    out_ref[...] = jnp.concatenate(
        [_expmap0(xc[:, :D]), _expmap0(xc[:, D:2 * D]), xc[:, 2 * D:]],
        axis=-1)


def _sc_gather(table, row, col):
    e = row.shape[0]
    fw = table.shape[1]
    nw = 32
    per = e // nw
    ch = 128
    nfull = per // ch
    rem = per - nfull * ch
    mesh = plsc.VectorSubcoreMesh(core_axis_name="c", subcore_axis_name="s")

    @functools.partial(
        pl.kernel, mesh=mesh,
        out_type=(jax.ShapeDtypeStruct((e, fw), jnp.float32),
                  jax.ShapeDtypeStruct((e, fw), jnp.float32)),
        scratch_types=[
            pltpu.VMEM((ch,), jnp.int32),
            pltpu.VMEM((ch, fw), jnp.float32),
            pltpu.VMEM((16,), jnp.int32),
            pltpu.VMEM((16, fw), jnp.float32),
            pltpu.SemaphoreType.DMA,
        ],
    )
    def gk(tab_h, row_h, col_h, r_h, c_h, idx_v, buf_v, idxs_v, bufs_v, sem):
        wid = lax.axis_index("s") * 2 + lax.axis_index("c")
        base = wid * per

        def do(idx_h, out_h):
            def body(j, carry):
                off = base + j * ch
                pltpu.sync_copy(idx_h.at[pl.ds(off, ch)], idx_v)
                pltpu.async_copy(tab_h.at[idx_v], buf_v, sem).wait()
                pltpu.sync_copy(buf_v, out_h.at[pl.ds(off, ch)])
                return carry
            lax.fori_loop(0, nfull, body, 0)
            if rem:
                off = base + per - 16
                pltpu.sync_copy(idx_h.at[pl.ds(off, 16)], idxs_v)
                pltpu.async_copy(tab_h.at[idxs_v], bufs_v, sem).wait()
                pltpu.sync_copy(bufs_v, out_h.at[pl.ds(off, 16)])

        do(row_h, r_h)
        do(col_h, c_h)

    return gk(table, row, col)


def _scatter_chunks(row_h, w_h, acc_s, idx_v, buf_v, idxt_v, buft_v,
                    base, ch, nfull, tail_cond, coloff):
    def sc(j, carry):
        off = base + j * ch
        pltpu.sync_copy(row_h.at[pl.ds(off, ch)], idx_v)
        pltpu.sync_copy(w_h.at[pl.ds(off, ch), pl.ds(coloff, D)], buf_v)
        pltpu.sync_copy(buf_v, acc_s.at[idx_v], add=True)
        return carry
    lax.fori_loop(0, nfull, sc, 0)

    @pl.when(tail_cond)
    def _():
        off = base + nfull * ch
        pltpu.sync_copy(row_h.at[pl.ds(off, 16)], idxt_v)
        pltpu.sync_copy(w_h.at[pl.ds(off, 16), pl.ds(coloff, D)], buft_v)
        pltpu.sync_copy(buft_v, acc_s.at[idxt_v], add=True)


def _zero_acc(z_h, blk_v, acc_s, s, zr, nzb, nzi):
    pltpu.sync_copy(z_h, blk_v)

    def zb(j, carry):
        b = j * 16 + s

        @pl.when(b < nzb)
        def _():
            pltpu.sync_copy(blk_v, acc_s.at[pl.ds(b * zr, zr)])
        return carry
    lax.fori_loop(0, nzi, zb, 0)


def _writeout(acc_s, out_h, blk_v, s, zr, nzb, nzi, coloff):
    def wb(j, carry):
        b = j * 16 + s

        @pl.when(b < nzb)
        def _():
            pltpu.sync_copy(acc_s.at[pl.ds(b * zr, zr)], blk_v)
            pltpu.sync_copy(
                blk_v, out_h.at[pl.ds(b * zr, zr), pl.ds(coloff, D)])
        return carry
    lax.fori_loop(0, nzi, wb, 0)


def _sc_scatter(wmsg, row, nnodes):
    e, fw = wmsg.shape
    ch = 128
    zr = 8
    nzb = nnodes // zr
    nzi = (nzb + 15) // 16
    zeros = jnp.zeros((zr, D), jnp.float32)
    mesh = plsc.VectorSubcoreMesh(core_axis_name="c", subcore_axis_name="s")
    scratch = [
        pltpu.VMEM((ch,), jnp.int32),
        pltpu.VMEM((ch, D), jnp.float32),
        pltpu.VMEM((16,), jnp.int32),
        pltpu.VMEM((16, D), jnp.float32),
        pltpu.VMEM((zr, D), jnp.float32),
        pltpu.VMEM_SHARED((nnodes, D), jnp.float32),
    ]

    pert1 = e // 16
    nf1 = pert1 // ch

    @functools.partial(
        pl.kernel, mesh=mesh,
        out_type=jax.ShapeDtypeStruct((nnodes, 2 * D), jnp.float32),
        scratch_types=scratch,
    )
    def sk1(w_h, row_h, z_h, agg_h, idx_v, buf_v, idxt_v, buft_v, blk_v,
            acc_s):
        c = lax.axis_index("c")
        s = lax.axis_index("s")
        _zero_acc(z_h, blk_v, acc_s, s, zr, nzb, nzi)
        plsc.subcore_barrier()
        _scatter_chunks(row_h, w_h, acc_s, idx_v, buf_v, idxt_v, buft_v,
                        s * pert1, ch, nf1, s >= 0, c * D)
        plsc.subcore_barrier()
        _writeout(acc_s, agg_h, blk_v, s, zr, nzb, nzi, c * D)

    nf2 = (e // 32) // ch

    @functools.partial(
        pl.kernel, mesh=mesh,
        out_type=jax.ShapeDtypeStruct((nnodes, 2 * D), jnp.float32),
        scratch_types=scratch,
    )
    def sk2(w_h, row_h, z_h, agg_h, idx_v, buf_v, idxt_v, buft_v, blk_v,
            acc_s):
        c = lax.axis_index("c")
        s = lax.axis_index("s")
        wid = c * 16 + s
        base = wid * (nf2 * ch) + jnp.minimum(wid, 16) * 16
        _zero_acc(z_h, blk_v, acc_s, s, zr, nzb, nzi)
        plsc.subcore_barrier()
        _scatter_chunks(row_h, w_h, acc_s, idx_v, buf_v, idxt_v, buft_v,
                        base, ch, nf2, wid < 16, 2 * D)
        plsc.subcore_barrier()
        _writeout(acc_s, agg_h, blk_v, s, zr, nzb, nzi, c * D)

    agg_a = sk1(wmsg, row, zeros)
    agg_b = sk2(wmsg, row, zeros)
    return agg_a, agg_b


def kernel(x, edge_attr, edges, node_mask, edge_mask, W_lin, bias, W_e1, b_e1,
           W_e2, b_e2, W_n1, b_n1, W_n2, b_n2, W_a1, b_a1, W_a2, b_a2,
           ln_g, ln_b):
    n = x.shape[0]
    e = edge_attr.shape[0]
    xf = x.reshape(n, F)
    bn = 1000

    t_tab, xtan = pl.pallas_call(
        _node_pre_body,
        grid=(n // bn,),
        in_specs=[
            pl.BlockSpec((bn, F), lambda i: (i, 0)),
            pl.BlockSpec((F, F), lambda i: (0, 0)),
            pl.BlockSpec((1, F), lambda i: (0, 0)),
        ],
        out_specs=[pl.BlockSpec((bn, F), lambda i: (i, 0))] * 2,
        out_shape=[jax.ShapeDtypeStruct((n, F), jnp.float32)] * 2,
    )(xf, W_lin.T, bias)

    row = edges[0]
    col = edges[1]
    r_tab, c_tab = _sc_gather(t_tab, row, col)

    wa1t = W_a1.T
    wa1t = wa1t.at[jnp.array([0, D, F, F + D])].set(0.0)

    be = 640
    wmsg, ea = pl.pallas_call(
        _edge_body,
        grid=(e // be,),
        in_specs=[
            pl.BlockSpec((be, F), lambda i: (i, 0)),
            pl.BlockSpec((be, F), lambda i: (i, 0)),
            pl.BlockSpec((be, 2), lambda i: (i, 0)),
            pl.BlockSpec((be, 1), lambda i: (i, 0)),
            pl.BlockSpec((2 * F + 4, F), lambda i: (0, 0)),
            pl.BlockSpec((1, F), lambda i: (0, 0)),
            pl.BlockSpec((F, 3), lambda i: (0, 0)),
            pl.BlockSpec((1, 3), lambda i: (0, 0)),
            pl.BlockSpec((F + 4, F), lambda i: (0, 0)),
            pl.BlockSpec((1, F), lambda i: (0, 0)),
            pl.BlockSpec((F, F), lambda i: (0, 0)),
            pl.BlockSpec((1, F), lambda i: (0, 0)),
        ],
        out_specs=[
            pl.BlockSpec((be, F), lambda i: (i, 0)),
            pl.BlockSpec((be, 4), lambda i: (i, 0)),
        ],
        out_shape=[
            jax.ShapeDtypeStruct((e, F), jnp.float32),
            jax.ShapeDtypeStruct((e, 4), jnp.float32),
        ],
    )(r_tab, c_tab, edge_attr, edge_mask, wa1t, b_a1.reshape(1, F),
      W_a2.T, b_a2.reshape(1, 3), W_e1.T, b_e1.reshape(1, F), W_e2.T,
      b_e2.reshape(1, F))

    agg_a, agg_b = _sc_scatter(wmsg, row, n)

    lng = jnp.concatenate([jnp.zeros((1,), jnp.float32), ln_g]).reshape(1, F)
    lnb = jnp.concatenate([jnp.zeros((1,), jnp.float32), ln_b]).reshape(1, F)
    out = pl.pallas_call(
        _node_post_body,
        grid=(n // bn,),
        in_specs=[
            pl.BlockSpec((bn, F), lambda i: (i, 0)),
            pl.BlockSpec((bn, F), lambda i: (i, 0)),
            pl.BlockSpec((bn, 2 * D), lambda i: (i, 0)),
            pl.BlockSpec((bn, 2 * D), lambda i: (i, 0)),
            pl.BlockSpec((2 * F, F), lambda i: (0, 0)),
            pl.BlockSpec((1, F), lambda i: (0, 0)),
            pl.BlockSpec((F, F), lambda i: (0, 0)),
            pl.BlockSpec((1, F), lambda i: (0, 0)),
            pl.BlockSpec((1, F), lambda i: (0, 0)),
            pl.BlockSpec((1, F), lambda i: (0, 0)),
        ],
        out_specs=[pl.BlockSpec((bn, F), lambda i: (i, 0))],
        out_shape=[jax.ShapeDtypeStruct((n, F), jnp.float32)],
    )(t_tab, xtan, agg_a, agg_b, W_n1.T, b_n1.reshape(1, F), W_n2.T,
      b_n2.reshape(1, F), lng, lnb)[0]

    return out.reshape(n, 3, D), ea, edges, node_mask, edge_mask


# speedup vs baseline: 1.8190x; 1.1888x over previous
import functools

import jax
import jax.numpy as jnp
from jax import lax
from jax.experimental import pallas as pl
from jax.experimental.pallas import tpu as pltpu
from jax.experimental.pallas import tpu_sc as plsc

EPS = 1e-7
D = 128
F = 384


def _col0_mask(c):
    return lax.broadcasted_iota(jnp.int32, (1, c), 1) == 0


def _zero_col0(a):
    return jnp.where(_col0_mask(a.shape[-1]), 0.0, a)


def _mdot(a, b):
    return jnp.sum(a * b, axis=-1, keepdims=True) - 2.0 * a[:, 0:1] * b[:, 0:1]


def _acosh(z):
    z = jnp.maximum(z, 1.0 + EPS)
    return jnp.log(z + jnp.sqrt((z - 1.0) * (z + 1.0)))


def _spn2(a):
    s = jnp.sum(a * a, axis=-1, keepdims=True) - a[:, 0:1] * a[:, 0:1]
    return jnp.maximum(s, 0.0)


def _logmap0(xp):
    d = _acosh(xp[:, 0:1])
    spn = jnp.sqrt(_spn2(xp) + 1e-15)
    return _zero_col0(xp * (d / spn))


def _expmap0(u):
    nrm = jnp.sqrt(_spn2(u) + 1e-15)
    e = jnp.exp(nrm)
    ei = 1.0 / e
    c = 0.5 * (e + ei)
    s = 0.5 * (e - ei) / nrm
    return jnp.where(_col0_mask(u.shape[-1]), c, u * s)


def _transp0(xp, u):
    f = _mdot(xp, u) / (1.0 + xp[:, 0:1])
    return u + f * xp + jnp.where(_col0_mask(xp.shape[-1]), f, 0.0)


def _expmap(xp, u):
    un = jnp.sqrt(jnp.maximum(_mdot(u, u), 1e-8))
    e = jnp.exp(un)
    ei = 1.0 / e
    return 0.5 * (e + ei) * xp + (0.5 * (e - ei) / un) * u


def _silu(z):
    return z / (1.0 + jnp.exp(-z))


def _sigmoid(z):
    return 1.0 / (1.0 + jnp.exp(-z))


def _node_pre_body(xf_ref, wlt_ref, bias_ref, t_ref, xtan_ref):
    xf = xf_ref[...]
    h = jnp.concatenate(
        [_logmap0(xf[:, :D]), _logmap0(xf[:, D:2 * D]), xf[:, 2 * D:]], axis=-1)
    h = jnp.dot(h, wlt_ref[...], preferred_element_type=jnp.float32)
    h = _zero_col0(h)
    x0p = _expmap0(h[:, :D])
    x1p = _expmap0(h[:, D:2 * D])
    x2 = h[:, 2 * D:]
    bias = _zero_col0(bias_ref[...])
    x0p = _expmap(x0p, _transp0(x0p, bias[:, :D]))
    x1p = _expmap(x1p, _transp0(x1p, bias[:, D:2 * D]))
    x2 = x2 + bias[:, 2 * D:]
    t_ref[...] = jnp.concatenate([x0p, x1p, x2], axis=-1)
    xtan_ref[...] = jnp.concatenate(
        [_logmap0(x0p), _logmap0(x1p), x2], axis=-1)


def _logmap0_h(xp):
    t = xp[:, 0:1]
    d = _acosh(t)
    spn = jnp.sqrt(jnp.maximum(t * t - 1.0, 0.0) + 1e-15)
    return xp * (d / spn)


def _edge_body(r_ref, c_ref, eattr_ref, emask_ref, wa1_ref, ba1_ref,
               wa2_ref, ba2_ref, we1_ref, be1_ref, we2_ref, be2_ref, wmsg_ref,
               ea_ref):
    rb = r_ref[...]
    cb = c_ref[...]
    x0r, x1r, x2r = rb[:, :D], rb[:, D:2 * D], rb[:, 2 * D:]
    x0c, x1c, x2c = cb[:, :D], cb[:, D:2 * D], cb[:, 2 * D:]
    md0 = _mdot(x0r, x0c)
    md1 = _mdot(x1r, x1c)
    geo0 = _acosh(-md0)
    geo1 = _acosh(-md1)
    em = emask_ref[...]
    ea = jnp.concatenate([eattr_ref[...], geo0, geo1], axis=-1)
    distm = ea * em
    xtr = jnp.concatenate([_logmap0_h(x0r), _logmap0_h(x1r), x2r], axis=-1)
    xtc = jnp.concatenate([_logmap0_h(x0c), _logmap0_h(x1c), x2c], axis=-1)
    att_in = jnp.concatenate([xtr, xtc, distm], axis=-1)
    a1 = _silu(jnp.dot(att_in, wa1_ref[...],
                       preferred_element_type=jnp.float32) + ba1_ref[...])
    att = _sigmoid(jnp.dot(a1, wa2_ref[...],
                           preferred_element_type=jnp.float32)
                   + ba2_ref[...]) * em

    u0 = x0c + md0 * x0r
    u1 = x1c + md1 * x1r
    uu0 = md0 * md0 - 1.0
    uu1 = md1 * md1 - 1.0

    def mcalc(xr, u, uu, geo):
        un = jnp.sqrt(jnp.maximum(uu, 1e-8))
        mu = (geo / un) * u
        cc = -mu[:, 0:1] / (1.0 + xr[:, 0:1])
        return mu + cc * xr + jnp.where(_col0_mask(D), cc, 0.0)

    m0 = mcalc(x0r, u0, uu0, geo0)
    m1 = mcalc(x1r, u1, uu1, geo1)
    m2 = x2c - x2r
    mi = jnp.concatenate([m0, m1, m2, ea], axis=-1)
    mm = _silu(jnp.dot(mi, we1_ref[...],
                       preferred_element_type=jnp.float32) + be1_ref[...])
    mm = jnp.dot(mm, we2_ref[...],
                 preferred_element_type=jnp.float32) + be2_ref[...]
    wmsg_ref[...] = jnp.concatenate(
        [mm[:, :D] * att[:, 0:1], mm[:, D:2 * D] * att[:, 1:2],
         mm[:, 2 * D:] * att[:, 2:3]], axis=-1)
    ea_ref[...] = ea


def _node_post_body(t_ref, xtan_ref, aa0_ref, aa1_ref, ab0_ref, ab1_ref,
                    wn1_ref, bn1_ref, wn2_ref, bn2_ref, lng_ref, lnb_ref,
                    out_ref):
    tb = t_ref[...]
    x0p, x1p, x2 = tb[:, :D], tb[:, D:2 * D], tb[:, 2 * D:]
    ab0 = ab0_ref[...]
    ab1 = ab1_ref[...]
    agg2 = ab0[:, :D] + ab0[:, D:] + ab1[:, :D] + ab1[:, D:]
    h = jnp.concatenate(
        [xtan_ref[...], aa0_ref[...] + aa1_ref[...], agg2], axis=-1)
    h = _silu(jnp.dot(h, wn1_ref[...],
                      preferred_element_type=jnp.float32) + bn1_ref[...])
    h = jnp.dot(h, wn2_ref[...],
                preferred_element_type=jnp.float32) + bn2_ref[...]
    h = _zero_col0(h)
    x0n = _expmap(x0p, _transp0(x0p, h[:, :D]))
    x1n = _expmap(x1p, _transp0(x1p, h[:, D:2 * D]))
    x2n = x2 + h[:, 2 * D:]
    xc = jnp.concatenate([_logmap0(x0n), _logmap0(x1n), x2n], axis=-1)
    t0 = xc[:, 0:1]
    mu = (jnp.sum(xc, axis=-1, keepdims=True) - t0) / (F - 1.0)
    dd = xc - mu
    d0 = t0 - mu
    var = (jnp.sum(dd * dd, axis=-1, keepdims=True) - d0 * d0) / (F - 1.0)
    tail = dd * lax.rsqrt(var + 1e-5) * lng_ref[...] + lnb_ref[...]
    xc = jnp.where(_col0_mask(F), xc, tail)
    xc = _zero_col0(_silu(xc))
    out_ref[...] = jnp.concatenate(
        [_expmap0(xc[:, :D]), _expmap0(xc[:, D:2 * D]), xc[:, 2 * D:]],
        axis=-1)


def _sc_gather(table, row, col):
    e = row.shape[0]
    fw = table.shape[1]
    qu, rm = divmod(e // 16, 32)
    ch = 96
    nfull = qu * 16 // ch
    assert nfull * ch == qu * 16
    mesh = plsc.VectorSubcoreMesh(core_axis_name="c", subcore_axis_name="s")

    @functools.partial(
        pl.kernel, mesh=mesh,
        out_type=(jax.ShapeDtypeStruct((e, fw), jnp.float32),
                  jax.ShapeDtypeStruct((e, fw), jnp.float32)),
        scratch_types=[
            pltpu.VMEM((ch,), jnp.int32),
            pltpu.VMEM((ch, fw), jnp.float32),
            pltpu.VMEM((16,), jnp.int32),
            pltpu.VMEM((16, fw), jnp.float32),
            pltpu.SemaphoreType.DMA,
        ],
    )
    def gk(tab_h, row_h, col_h, r_h, c_h, idx_v, buf_v, idxs_v, bufs_v, sem):
        wid = lax.axis_index("s") * 2 + lax.axis_index("c")
        base = (wid * qu + jnp.minimum(wid, rm)) * 16

        def do(idx_h, out_h):
            def body(j, carry):
                off = base + j * ch
                pltpu.sync_copy(idx_h.at[pl.ds(off, ch)], idx_v)
                pltpu.async_copy(tab_h.at[idx_v], buf_v, sem).wait()
                pltpu.sync_copy(buf_v, out_h.at[pl.ds(off, ch)])
                return carry
            lax.fori_loop(0, nfull, body, 0)

            @pl.when(wid < rm)
            def _():
                off = base + nfull * ch
                pltpu.sync_copy(idx_h.at[pl.ds(off, 16)], idxs_v)
                pltpu.async_copy(tab_h.at[idxs_v], bufs_v, sem).wait()
                pltpu.sync_copy(bufs_v, out_h.at[pl.ds(off, 16)])

        do(row_h, r_h)
        do(col_h, c_h)

    return gk(table, row, col)


def _scatter_chunks(row_h, w_h, acc_s, idx_v, buf_v, idxt_v, buft_v,
                    base, ch, nfull, tail_cond, coloff):
    def sc(j, carry):
        off = base + j * ch
        pltpu.sync_copy(row_h.at[pl.ds(off, ch)], idx_v)
        pltpu.sync_copy(w_h.at[pl.ds(off, ch), pl.ds(coloff, D)], buf_v)
        pltpu.sync_copy(buf_v, acc_s.at[idx_v], add=True)
        return carry
    lax.fori_loop(0, nfull, sc, 0)

    @pl.when(tail_cond)
    def _():
        off = base + nfull * ch
        pltpu.sync_copy(row_h.at[pl.ds(off, 16)], idxt_v)
        pltpu.sync_copy(w_h.at[pl.ds(off, 16), pl.ds(coloff, D)], buft_v)
        pltpu.sync_copy(buft_v, acc_s.at[idxt_v], add=True)


def _zero_acc(z_h, blk_v, acc_s, s, zr, nzb, nzi):
    pltpu.sync_copy(z_h, blk_v)

    def zb(j, carry):
        b = j * 16 + s

        @pl.when(b < nzb)
        def _():
            pltpu.sync_copy(blk_v, acc_s.at[pl.ds(b * zr, zr)])
        return carry
    lax.fori_loop(0, nzi, zb, 0)


def _writeout(acc_s, out_h, blk_v, s, zr, nzb, nzi, coloff):
    def wb(j, carry):
        b = j * 16 + s

        @pl.when(b < nzb)
        def _():
            pltpu.sync_copy(acc_s.at[pl.ds(b * zr, zr)], blk_v)
            pltpu.sync_copy(
                blk_v, out_h.at[pl.ds(b * zr, zr), pl.ds(coloff, D)])
        return carry
    lax.fori_loop(0, nzi, wb, 0)


def _sc_scatter(wmsg, row, nnodes):
    e, fw = wmsg.shape
    zr = 8
    nzb = nnodes // zr
    nzi = (nzb + 15) // 16
    zeros = jnp.zeros((zr, D), jnp.float32)
    mesh = plsc.VectorSubcoreMesh(core_axis_name="c", subcore_axis_name="s")

    def make_launch(ntiles, ch, coloff_fn):
        qu, rm = divmod(e // 16, ntiles)
        nfull = qu * 16 // ch
        assert nfull * ch == qu * 16
        scratch = [
            pltpu.VMEM((ch,), jnp.int32),
            pltpu.VMEM((ch, D), jnp.float32),
            pltpu.VMEM((16,), jnp.int32),
            pltpu.VMEM((16, D), jnp.float32),
            pltpu.VMEM((zr, D), jnp.float32),
            pltpu.VMEM_SHARED((nnodes, D), jnp.float32),
        ]

        @functools.partial(
            pl.kernel, mesh=mesh,
            out_type=jax.ShapeDtypeStruct((nnodes, 2 * D), jnp.float32),
            scratch_types=scratch,
        )
        def sk(w_h, row_h, z_h, agg_h, idx_v, buf_v, idxt_v, buft_v, blk_v,
               acc_s):
            c = lax.axis_index("c")
            s = lax.axis_index("s")
            tid = s if ntiles == 16 else c * 16 + s
            base = (tid * qu + jnp.minimum(tid, rm)) * 16
            _zero_acc(z_h, blk_v, acc_s, s, zr, nzb, nzi)
            plsc.subcore_barrier()
            _scatter_chunks(row_h, w_h, acc_s, idx_v, buf_v, idxt_v, buft_v,
                            base, ch, nfull, tid < rm, coloff_fn(c))
            plsc.subcore_barrier()
            _writeout(acc_s, agg_h, blk_v, s, zr, nzb, nzi, c * D)

        return sk

    agg_a = make_launch(16, 128, lambda c: c * D)(wmsg, row, zeros)
    agg_b = make_launch(32, 96, lambda c: 2 * D)(wmsg, row, zeros)
    return agg_a, agg_b


def kernel(x, edge_attr, edges, node_mask, edge_mask, W_lin, bias, W_e1, b_e1,
           W_e2, b_e2, W_n1, b_n1, W_n2, b_n2, W_a1, b_a1, W_a2, b_a2,
           ln_g, ln_b):
    n = x.shape[0]
    e = edge_attr.shape[0]
    xf = x.reshape(n, F)
    bn = 1000

    t_tab, xtan = pl.pallas_call(
        _node_pre_body,
        grid=(n // bn,),
        in_specs=[
            pl.BlockSpec((bn, F), lambda i: (i, 0)),
            pl.BlockSpec((F, F), lambda i: (0, 0)),
            pl.BlockSpec((1, F), lambda i: (0, 0)),
        ],
        out_specs=[pl.BlockSpec((bn, F), lambda i: (i, 0))] * 2,
        out_shape=[jax.ShapeDtypeStruct((n, F), jnp.float32)] * 2,
    )(xf, W_lin.T, bias)

    row = edges[0]
    col = edges[1]

    wa1t = W_a1.T
    wa1t = wa1t.at[jnp.array([0, D, F, F + D])].set(0.0)

    be = 640
    eh = e // 2

    def edge_half(h):
        sl = slice(h * eh, (h + 1) * eh)
        r_tab, c_tab = _sc_gather(t_tab, row[sl], col[sl])
        wmsg, ea = pl.pallas_call(
            _edge_body,
            grid=(eh // be,),
            in_specs=[
                pl.BlockSpec((be, F), lambda i: (i, 0)),
                pl.BlockSpec((be, F), lambda i: (i, 0)),
                pl.BlockSpec((be, 2), lambda i: (i, 0)),
                pl.BlockSpec((be, 1), lambda i: (i, 0)),
                pl.BlockSpec((2 * F + 4, F), lambda i: (0, 0)),
                pl.BlockSpec((1, F), lambda i: (0, 0)),
                pl.BlockSpec((F, 3), lambda i: (0, 0)),
                pl.BlockSpec((1, 3), lambda i: (0, 0)),
                pl.BlockSpec((F + 4, F), lambda i: (0, 0)),
                pl.BlockSpec((1, F), lambda i: (0, 0)),
                pl.BlockSpec((F, F), lambda i: (0, 0)),
                pl.BlockSpec((1, F), lambda i: (0, 0)),
            ],
            out_specs=[
                pl.BlockSpec((be, F), lambda i: (i, 0)),
                pl.BlockSpec((be, 4), lambda i: (i, 0)),
            ],
            out_shape=[
                jax.ShapeDtypeStruct((eh, F), jnp.float32),
                jax.ShapeDtypeStruct((eh, 4), jnp.float32),
            ],
        )(r_tab, c_tab, edge_attr[sl], edge_mask[sl], wa1t,
          b_a1.reshape(1, F), W_a2.T, b_a2.reshape(1, 3), W_e1.T,
          b_e1.reshape(1, F), W_e2.T, b_e2.reshape(1, F))
        agg_a, agg_b = _sc_scatter(wmsg, row[sl], n)
        return ea, agg_a, agg_b

    ea0, agg_a0, agg_b0 = edge_half(0)
    ea1, agg_a1, agg_b1 = edge_half(1)
    ea = jnp.concatenate([ea0, ea1], axis=0)

    lng = jnp.concatenate([jnp.zeros((1,), jnp.float32), ln_g]).reshape(1, F)
    lnb = jnp.concatenate([jnp.zeros((1,), jnp.float32), ln_b]).reshape(1, F)
    out = pl.pallas_call(
        _node_post_body,
        grid=(n // bn,),
        in_specs=[
            pl.BlockSpec((bn, F), lambda i: (i, 0)),
            pl.BlockSpec((bn, F), lambda i: (i, 0)),
            pl.BlockSpec((bn, 2 * D), lambda i: (i, 0)),
            pl.BlockSpec((bn, 2 * D), lambda i: (i, 0)),
            pl.BlockSpec((bn, 2 * D), lambda i: (i, 0)),
            pl.BlockSpec((bn, 2 * D), lambda i: (i, 0)),
            pl.BlockSpec((2 * F, F), lambda i: (0, 0)),
            pl.BlockSpec((1, F), lambda i: (0, 0)),
            pl.BlockSpec((F, F), lambda i: (0, 0)),
            pl.BlockSpec((1, F), lambda i: (0, 0)),
            pl.BlockSpec((1, F), lambda i: (0, 0)),
            pl.BlockSpec((1, F), lambda i: (0, 0)),
        ],
        out_specs=[pl.BlockSpec((bn, F), lambda i: (i, 0))],
        out_shape=[jax.ShapeDtypeStruct((n, F), jnp.float32)],
    )(t_tab, xtan, agg_a0, agg_a1, agg_b0, agg_b1, W_n1.T,
      b_n1.reshape(1, F), W_n2.T, b_n2.reshape(1, F), lng, lnb)[0]

    return out.reshape(n, 3, D), ea, edges, node_mask, edge_mask


# speedup vs baseline: 1.9208x; 1.0559x over previous
import functools

import jax
import jax.numpy as jnp
from jax import lax
from jax.experimental import pallas as pl
from jax.experimental.pallas import tpu as pltpu
from jax.experimental.pallas import tpu_sc as plsc

EPS = 1e-7
D = 128
F = 384


def _col0_mask(c):
    return lax.broadcasted_iota(jnp.int32, (1, c), 1) == 0


def _zero_col0(a):
    return jnp.where(_col0_mask(a.shape[-1]), 0.0, a)


def _mdot(a, b):
    return jnp.sum(a * b, axis=-1, keepdims=True) - 2.0 * a[:, 0:1] * b[:, 0:1]


def _acosh(z):
    z = jnp.maximum(z, 1.0 + EPS)
    return jnp.log(z + jnp.sqrt((z - 1.0) * (z + 1.0)))


def _spn2(a):
    s = jnp.sum(a * a, axis=-1, keepdims=True) - a[:, 0:1] * a[:, 0:1]
    return jnp.maximum(s, 0.0)


def _logmap0(xp):
    d = _acosh(xp[:, 0:1])
    spn = jnp.sqrt(_spn2(xp) + 1e-15)
    return _zero_col0(xp * (d / spn))


def _expmap0(u):
    nrm = jnp.sqrt(_spn2(u) + 1e-15)
    e = jnp.exp(nrm)
    ei = 1.0 / e
    c = 0.5 * (e + ei)
    s = 0.5 * (e - ei) / nrm
    return jnp.where(_col0_mask(u.shape[-1]), c, u * s)


def _transp0(xp, u):
    f = _mdot(xp, u) / (1.0 + xp[:, 0:1])
    return u + f * xp + jnp.where(_col0_mask(xp.shape[-1]), f, 0.0)


def _expmap(xp, u):
    un = jnp.sqrt(jnp.maximum(_mdot(u, u), 1e-8))
    e = jnp.exp(un)
    ei = 1.0 / e
    return 0.5 * (e + ei) * xp + (0.5 * (e - ei) / un) * u


def _silu(z):
    return z / (1.0 + jnp.exp(-z))


def _sigmoid(z):
    return 1.0 / (1.0 + jnp.exp(-z))


def _node_pre_body(xf_ref, wlt_ref, bias_ref, t_ref, xtan_ref):
    xf = xf_ref[...]
    h = jnp.concatenate(
        [_logmap0(xf[:, :D]), _logmap0(xf[:, D:2 * D]), xf[:, 2 * D:]], axis=-1)
    h = jnp.dot(h, wlt_ref[...], preferred_element_type=jnp.float32)
    h = _zero_col0(h)
    x0p = _expmap0(h[:, :D])
    x1p = _expmap0(h[:, D:2 * D])
    x2 = h[:, 2 * D:]
    bias = _zero_col0(bias_ref[...])
    x0p = _expmap(x0p, _transp0(x0p, bias[:, :D]))
    x1p = _expmap(x1p, _transp0(x1p, bias[:, D:2 * D]))
    x2 = x2 + bias[:, 2 * D:]
    t_ref[...] = jnp.concatenate([x0p, x1p, x2], axis=-1)
    xtan_ref[...] = jnp.concatenate(
        [_logmap0(x0p), _logmap0(x1p), x2], axis=-1)


def _logmap0_h(xp):
    t = xp[:, 0:1]
    d = _acosh(t)
    spn = jnp.sqrt(jnp.maximum(t * t - 1.0, 0.0) + 1e-15)
    return xp * (d / spn)


def _edge_body(r_ref, c_ref, eattr_ref, emask_ref, wa1_ref, ba1_ref,
               wa2_ref, ba2_ref, we1_ref, be1_ref, we2_ref, be2_ref, wmsg_ref,
               ea_ref):
    rb = r_ref[...]
    cb = c_ref[...]
    x0r, x1r, x2r = rb[:, :D], rb[:, D:2 * D], rb[:, 2 * D:]
    x0c, x1c, x2c = cb[:, :D], cb[:, D:2 * D], cb[:, 2 * D:]
    md0 = _mdot(x0r, x0c)
    md1 = _mdot(x1r, x1c)
    geo0 = _acosh(-md0)
    geo1 = _acosh(-md1)
    em = emask_ref[...]
    ea = jnp.concatenate([eattr_ref[...], geo0, geo1], axis=-1)
    distm = ea * em
    xtr = jnp.concatenate([_logmap0_h(x0r), _logmap0_h(x1r), x2r], axis=-1)
    xtc = jnp.concatenate([_logmap0_h(x0c), _logmap0_h(x1c), x2c], axis=-1)
    att_in = jnp.concatenate([xtr, xtc, distm], axis=-1)
    a1 = _silu(jnp.dot(att_in, wa1_ref[...],
                       preferred_element_type=jnp.float32) + ba1_ref[...])
    att = _sigmoid(jnp.dot(a1, wa2_ref[...],
                           preferred_element_type=jnp.float32)
                   + ba2_ref[...]) * em

    u0 = x0c + md0 * x0r
    u1 = x1c + md1 * x1r
    uu0 = md0 * md0 - 1.0
    uu1 = md1 * md1 - 1.0

    def mcalc(xr, u, uu, geo):
        un = jnp.sqrt(jnp.maximum(uu, 1e-8))
        mu = (geo / un) * u
        cc = -mu[:, 0:1] / (1.0 + xr[:, 0:1])
        return mu + cc * xr + jnp.where(_col0_mask(D), cc, 0.0)

    m0 = mcalc(x0r, u0, uu0, geo0)
    m1 = mcalc(x1r, u1, uu1, geo1)
    m2 = x2c - x2r
    mi = jnp.concatenate([m0, m1, m2, ea], axis=-1)
    mm = _silu(jnp.dot(mi, we1_ref[...],
                       preferred_element_type=jnp.float32) + be1_ref[...])
    mm = jnp.dot(mm, we2_ref[...],
                 preferred_element_type=jnp.float32) + be2_ref[...]
    wmsg_ref[...] = jnp.concatenate(
        [mm[:, :D] * att[:, 0:1], mm[:, D:2 * D] * att[:, 1:2],
         mm[:, 2 * D:] * att[:, 2:3]], axis=-1)
    ea_ref[...] = ea


def _node_post_body(t_ref, xtan_ref, wn1_ref, bn1_ref, wn2_ref, bn2_ref,
                    lng_ref, lnb_ref, *agg_and_out):
    agg_refs = agg_and_out[:-1]
    out_ref = agg_and_out[-1]
    npart = len(agg_refs) // 2
    tb = t_ref[...]
    x0p, x1p, x2 = tb[:, :D], tb[:, D:2 * D], tb[:, 2 * D:]
    agg_a = agg_refs[0][...]
    for r in agg_refs[1:npart]:
        agg_a = agg_a + r[...]
    agg2 = None
    for r in agg_refs[npart:]:
        ab = r[...]
        part = ab[:, :D] + ab[:, D:]
        agg2 = part if agg2 is None else agg2 + part
    h = jnp.concatenate([xtan_ref[...], agg_a, agg2], axis=-1)
    h = _silu(jnp.dot(h, wn1_ref[...],
                      preferred_element_type=jnp.float32) + bn1_ref[...])
    h = jnp.dot(h, wn2_ref[...],
                preferred_element_type=jnp.float32) + bn2_ref[...]
    h = _zero_col0(h)
    x0n = _expmap(x0p, _transp0(x0p, h[:, :D]))
    x1n = _expmap(x1p, _transp0(x1p, h[:, D:2 * D]))
    x2n = x2 + h[:, 2 * D:]
    xc = jnp.concatenate([_logmap0(x0n), _logmap0(x1n), x2n], axis=-1)
    t0 = xc[:, 0:1]
    mu = (jnp.sum(xc, axis=-1, keepdims=True) - t0) / (F - 1.0)
    dd = xc - mu
    d0 = t0 - mu
    var = (jnp.sum(dd * dd, axis=-1, keepdims=True) - d0 * d0) / (F - 1.0)
    tail = dd * lax.rsqrt(var + 1e-5) * lng_ref[...] + lnb_ref[...]
    xc = jnp.where(_col0_mask(F), xc, tail)
    xc = _zero_col0(_silu(xc))
    out_ref[...] = jnp.concatenate(
        [_expmap0(xc[:, :D]), _expmap0(xc[:, D:2 * D]), xc[:, 2 * D:]],
        axis=-1)


def _sc_gather(table, row, col):
    e = row.shape[0]
    fw = table.shape[1]
    qu, rm = divmod(e // 16, 32)
    ch = 96
    nfull = qu * 16 // ch
    assert nfull * ch == qu * 16
    mesh = plsc.VectorSubcoreMesh(core_axis_name="c", subcore_axis_name="s")

    @functools.partial(
        pl.kernel, mesh=mesh,
        out_type=(jax.ShapeDtypeStruct((e, fw), jnp.float32),
                  jax.ShapeDtypeStruct((e, fw), jnp.float32)),
        scratch_types=[
            pltpu.VMEM((ch,), jnp.int32),
            pltpu.VMEM((ch, fw), jnp.float32),
            pltpu.VMEM((16,), jnp.int32),
            pltpu.VMEM((16, fw), jnp.float32),
            pltpu.SemaphoreType.DMA,
        ],
    )
    def gk(tab_h, row_h, col_h, r_h, c_h, idx_v, buf_v, idxs_v, bufs_v, sem):
        wid = lax.axis_index("s") * 2 + lax.axis_index("c")
        base = (wid * qu + jnp.minimum(wid, rm)) * 16

        def do(idx_h, out_h):
            def body(j, carry):
                off = base + j * ch
                pltpu.sync_copy(idx_h.at[pl.ds(off, ch)], idx_v)
                pltpu.async_copy(tab_h.at[idx_v], buf_v, sem).wait()
                pltpu.sync_copy(buf_v, out_h.at[pl.ds(off, ch)])
                return carry
            lax.fori_loop(0, nfull, body, 0)

            @pl.when(wid < rm)
            def _():
                off = base + nfull * ch
                pltpu.sync_copy(idx_h.at[pl.ds(off, 16)], idxs_v)
                pltpu.async_copy(tab_h.at[idxs_v], bufs_v, sem).wait()
                pltpu.sync_copy(bufs_v, out_h.at[pl.ds(off, 16)])

        do(row_h, r_h)
        do(col_h, c_h)

    return gk(table, row, col)


def _scatter_chunks(row_h, w_h, acc_s, idx_v, buf_v, idxt_v, buft_v,
                    base, ch, nfull, tail_cond, coloff):
    def sc(j, carry):
        off = base + j * ch
        pltpu.sync_copy(row_h.at[pl.ds(off, ch)], idx_v)
        pltpu.sync_copy(w_h.at[pl.ds(off, ch), pl.ds(coloff, D)], buf_v)
        pltpu.sync_copy(buf_v, acc_s.at[idx_v], add=True)
        return carry
    lax.fori_loop(0, nfull, sc, 0)

    @pl.when(tail_cond)
    def _():
        off = base + nfull * ch
        pltpu.sync_copy(row_h.at[pl.ds(off, 16)], idxt_v)
        pltpu.sync_copy(w_h.at[pl.ds(off, 16), pl.ds(coloff, D)], buft_v)
        pltpu.sync_copy(buft_v, acc_s.at[idxt_v], add=True)


def _zero_acc(z_h, blk_v, acc_s, s, zr, nzb, nzi):
    pltpu.sync_copy(z_h, blk_v)

    def zb(j, carry):
        b = j * 16 + s

        @pl.when(b < nzb)
        def _():
            pltpu.sync_copy(blk_v, acc_s.at[pl.ds(b * zr, zr)])
        return carry
    lax.fori_loop(0, nzi, zb, 0)


def _writeout(acc_s, out_h, blk_v, s, zr, nzb, nzi, coloff):
    def wb(j, carry):
        b = j * 16 + s

        @pl.when(b < nzb)
        def _():
            pltpu.sync_copy(acc_s.at[pl.ds(b * zr, zr)], blk_v)
            pltpu.sync_copy(
                blk_v, out_h.at[pl.ds(b * zr, zr), pl.ds(coloff, D)])
        return carry
    lax.fori_loop(0, nzi, wb, 0)


def _sc_scatter(wmsg, row, nnodes):
    e, fw = wmsg.shape
    zr = 8
    nzb = nnodes // zr
    nzi = (nzb + 15) // 16
    zeros = jnp.zeros((zr, D), jnp.float32)
    mesh = plsc.VectorSubcoreMesh(core_axis_name="c", subcore_axis_name="s")

    def make_launch(ntiles, ch, coloff_fn):
        qu, rm = divmod(e // 16, ntiles)
        nfull = qu * 16 // ch
        assert nfull * ch == qu * 16
        scratch = [
            pltpu.VMEM((ch,), jnp.int32),
            pltpu.VMEM((ch, D), jnp.float32),
            pltpu.VMEM((16,), jnp.int32),
            pltpu.VMEM((16, D), jnp.float32),
            pltpu.VMEM((zr, D), jnp.float32),
            pltpu.VMEM_SHARED((nnodes, D), jnp.float32),
        ]

        @functools.partial(
            pl.kernel, mesh=mesh,
            out_type=jax.ShapeDtypeStruct((nnodes, 2 * D), jnp.float32),
            scratch_types=scratch,
        )
        def sk(w_h, row_h, z_h, agg_h, idx_v, buf_v, idxt_v, buft_v, blk_v,
               acc_s):
            c = lax.axis_index("c")
            s = lax.axis_index("s")
            tid = s if ntiles == 16 else c * 16 + s
            base = (tid * qu + jnp.minimum(tid, rm)) * 16
            _zero_acc(z_h, blk_v, acc_s, s, zr, nzb, nzi)
            plsc.subcore_barrier()
            _scatter_chunks(row_h, w_h, acc_s, idx_v, buf_v, idxt_v, buft_v,
                            base, ch, nfull, tid < rm, coloff_fn(c))
            plsc.subcore_barrier()
            _writeout(acc_s, agg_h, blk_v, s, zr, nzb, nzi, c * D)

        return sk

    agg_a = make_launch(16, 96, lambda c: c * D)(wmsg, row, zeros)
    agg_b = make_launch(32, 96, lambda c: 2 * D)(wmsg, row, zeros)
    return agg_a, agg_b


def kernel(x, edge_attr, edges, node_mask, edge_mask, W_lin, bias, W_e1, b_e1,
           W_e2, b_e2, W_n1, b_n1, W_n2, b_n2, W_a1, b_a1, W_a2, b_a2,
           ln_g, ln_b):
    n = x.shape[0]
    e = edge_attr.shape[0]
    xf = x.reshape(n, F)
    bn = 1000

    t_tab, xtan = pl.pallas_call(
        _node_pre_body,
        grid=(n // bn,),
        in_specs=[
            pl.BlockSpec((bn, F), lambda i: (i, 0)),
            pl.BlockSpec((F, F), lambda i: (0, 0)),
            pl.BlockSpec((1, F), lambda i: (0, 0)),
        ],
        out_specs=[pl.BlockSpec((bn, F), lambda i: (i, 0))] * 2,
        out_shape=[jax.ShapeDtypeStruct((n, F), jnp.float32)] * 2,
    )(xf, W_lin.T, bias)

    row = edges[0]
    col = edges[1]

    wa1t = W_a1.T
    wa1t = wa1t.at[jnp.array([0, D, F, F + D])].set(0.0)

    be = 800
    npart = 4
    eh = e // npart

    def edge_half(h):
        sl = slice(h * eh, (h + 1) * eh)
        r_tab, c_tab = _sc_gather(t_tab, row[sl], col[sl])
        wmsg, ea = pl.pallas_call(
            _edge_body,
            grid=(eh // be,),
            in_specs=[
                pl.BlockSpec((be, F), lambda i: (i, 0)),
                pl.BlockSpec((be, F), lambda i: (i, 0)),
                pl.BlockSpec((be, 2), lambda i: (i, 0)),
                pl.BlockSpec((be, 1), lambda i: (i, 0)),
                pl.BlockSpec((2 * F + 4, F), lambda i: (0, 0)),
                pl.BlockSpec((1, F), lambda i: (0, 0)),
                pl.BlockSpec((F, 3), lambda i: (0, 0)),
                pl.BlockSpec((1, 3), lambda i: (0, 0)),
                pl.BlockSpec((F + 4, F), lambda i: (0, 0)),
                pl.BlockSpec((1, F), lambda i: (0, 0)),
                pl.BlockSpec((F, F), lambda i: (0, 0)),
                pl.BlockSpec((1, F), lambda i: (0, 0)),
            ],
            out_specs=[
                pl.BlockSpec((be, F), lambda i: (i, 0)),
                pl.BlockSpec((be, 4), lambda i: (i, 0)),
            ],
            out_shape=[
                jax.ShapeDtypeStruct((eh, F), jnp.float32),
                jax.ShapeDtypeStruct((eh, 4), jnp.float32),
            ],
        )(r_tab, c_tab, edge_attr[sl], edge_mask[sl], wa1t,
          b_a1.reshape(1, F), W_a2.T, b_a2.reshape(1, 3), W_e1.T,
          b_e1.reshape(1, F), W_e2.T, b_e2.reshape(1, F))
        agg_a, agg_b = _sc_scatter(wmsg, row[sl], n)
        return ea, agg_a, agg_b

    parts = [edge_half(h) for h in range(npart)]
    ea = jnp.concatenate([p[0] for p in parts], axis=0)
    agg_as = [p[1] for p in parts]
    agg_bs = [p[2] for p in parts]

    lng = jnp.concatenate([jnp.zeros((1,), jnp.float32), ln_g]).reshape(1, F)
    lnb = jnp.concatenate([jnp.zeros((1,), jnp.float32), ln_b]).reshape(1, F)
    out = pl.pallas_call(
        _node_post_body,
        grid=(n // bn,),
        in_specs=[
            pl.BlockSpec((bn, F), lambda i: (i, 0)),
            pl.BlockSpec((bn, F), lambda i: (i, 0)),
            pl.BlockSpec((2 * F, F), lambda i: (0, 0)),
            pl.BlockSpec((1, F), lambda i: (0, 0)),
            pl.BlockSpec((F, F), lambda i: (0, 0)),
            pl.BlockSpec((1, F), lambda i: (0, 0)),
            pl.BlockSpec((1, F), lambda i: (0, 0)),
            pl.BlockSpec((1, F), lambda i: (0, 0)),
        ] + [pl.BlockSpec((bn, 2 * D), lambda i: (i, 0))] * (2 * npart),
        out_specs=[pl.BlockSpec((bn, F), lambda i: (i, 0))],
        out_shape=[jax.ShapeDtypeStruct((n, F), jnp.float32)],
    )(t_tab, xtan, W_n1.T, b_n1.reshape(1, F), W_n2.T, b_n2.reshape(1, F),
      lng, lnb, *agg_as, *agg_bs)[0]

    return out.reshape(n, 3, D), ea, edges, node_mask, edge_mask
